# baseline (speedup 1.0000x reference)
import functools

import jax
import jax.numpy as jnp
from jax import lax
from jax.experimental import pallas as pl
from jax.experimental.pallas import tpu as pltpu
from jax.experimental.pallas import tpu_sc as plsc

NUM_FIELDS = 26
FIELD_DIM = 38461
D = 16
B = 4096
NNZ = 26
NC = 2
NS = 16
NW = NC * NS
ROWS_PT = B // NW
E_PT = ROWS_PT * NNZ
NV = E_PT // 16
CH = 128
NCH = E_PT // CH
OUT_PT = ROWS_PT * (NUM_FIELDS - 1)
SC_ROWS = NS * OUT_PT


@functools.partial(
    pl.kernel,
    out_type=jax.ShapeDtypeStruct((B * (NUM_FIELDS - 1), D), jnp.float32),
    mesh=plsc.VectorSubcoreMesh(core_axis_name="c", subcore_axis_name="s"),
    compiler_params=pltpu.CompilerParams(use_tc_tiling_on_sc=False,
                                         needs_layout_passes=False),
    scratch_types=[
        pltpu.VMEM((E_PT,), jnp.int32),
        pltpu.VMEM((E_PT,), jnp.int32),
        pltpu.VMEM((E_PT,), jnp.int32),
        pltpu.VMEM((NCH, CH), jnp.int32),
        pltpu.VMEM((NCH, CH), jnp.int32),
        pltpu.VMEM((E_PT, D), jnp.float32),
        pltpu.VMEM_SHARED((SC_ROWS, D), jnp.float32),
        pltpu.SemaphoreType.DMA,
    ],
)
def _emb(xf_hbm, xx_hbm, table_hbm, dbase_hbm, zeros_hbm, out_hbm, f_v, x_v,
         db_v, gidx_v, d_v, rows_v, acc_sh, sem):
    sid = lax.axis_index("s")
    wid = sid * NC + lax.axis_index("c")
    ebase = wid * E_PT
    pltpu.sync_copy(xf_hbm.at[pl.ds(ebase, E_PT)], f_v)
    pltpu.sync_copy(xx_hbm.at[pl.ds(ebase, E_PT)], x_v)
    pltpu.sync_copy(dbase_hbm, db_v)

    obase = sid * OUT_PT
    for v in range(NV):
        f = f_v[pl.ds(v * 16, 16)]
        xv = x_v[pl.ds(v * 16, 16)]
        nz = jnp.minimum(f, 1)
        gid = (xv + f * FIELD_DIM) * nz
        d = obase + db_v[pl.ds(v * 16, 16)] + f - nz
        gidx_v[v // 8, pl.ds((v % 8) * 16, 16)] = gid
        d_v[v // 8, pl.ds((v % 8) * 16, 16)] = d

    gathers = [
        pltpu.async_copy(table_hbm.at[gidx_v.at[j]],
                         rows_v.at[pl.ds(j * CH, CH)], sem)
        for j in range(NCH)
    ]

    pltpu.sync_copy(zeros_hbm, acc_sh.at[pl.ds(obase, OUT_PT)])

    for g in gathers:
        g.wait()

    adds = [
        pltpu.async_copy(rows_v.at[pl.ds(j * CH, CH)], acc_sh.at[d_v.at[j]],
                         sem, add=True)
        for j in range(NCH)
    ]
    for a in adds:
        a.wait()

    pltpu.sync_copy(acc_sh.at[pl.ds(obase, OUT_PT)],
                    out_hbm.at[pl.ds(wid * OUT_PT, OUT_PT)])


def kernel(x_field, x, table):
    xf = x_field.reshape(-1).astype(jnp.int32)
    xx = x.reshape(-1).astype(jnp.int32)
    dbase = (jnp.arange(E_PT, dtype=jnp.int32) // NNZ) * (NUM_FIELDS - 1)
    zeros = jnp.zeros((OUT_PT, D), jnp.float32)
    out = _emb(xf, xx, table, dbase, zeros)
    return out.reshape(B, NUM_FIELDS - 1, D)
